# baseline (speedup 1.0000x reference)
import jax
import jax.numpy as jnp
from jax.experimental import pallas as pl

N = 10000
H = 8
C = 128


def _gat_layer(x, src, dst, eaw, Wl, Wr, att, b):
    xl = (x @ Wl).reshape(-1, H, C)
    xr = (x @ Wr).reshape(-1, H, C)
    e = xl[src] + xr[dst] + eaw
    e = jax.nn.leaky_relu(e, 0.2)
    logits = jnp.sum(e * att[None, :, :], axis=-1)
    p = jnp.exp(logits)
    denom = jax.ops.segment_sum(p, dst, num_segments=N)
    msg = xl[src] * p[:, :, None]
    out = jax.ops.segment_sum(msg, dst, num_segments=N)
    out = out / (denom[:, :, None] + 1e-16)
    return out.reshape(-1, H * C) + b


def kernel(x, edge_index, edge_attr, batch, Wl1, Wr1, We1, att1, b1, Wl2, Wr2, We2, att2, b2, Wf1, bf1, Wf2, bf2):
    src = edge_index[0]
    dst = edge_index[1]
    order = jnp.argsort(dst)
    src_s = src[order]
    dst_s = dst[order]
    ea_s = edge_attr[order]
    starts = jnp.searchsorted(dst_s, jnp.arange(0, 10240, 32))
    src_s = src_s + 0 * starts[0]
    eaw1 = (ea_s @ We1).reshape(-1, H, C)
    eaw2 = (ea_s @ We2).reshape(-1, H, C)
    h = jax.nn.relu(_gat_layer(x, src_s, dst_s, eaw1, Wl1, Wr1, att1, b1))
    h = jax.nn.relu(_gat_layer(h, src_s, dst_s, eaw2, Wl2, Wr2, att2, b2))
    pooled = jnp.sum(h, axis=0, keepdims=True) / N
    z = jax.nn.relu(pooled @ Wf1 + bf1)
    return z @ Wf2 + bf2


# speedup vs baseline: 4.3441x; 4.3441x over previous
import functools

import jax
import jax.numpy as jnp
from jax import lax
from jax.experimental import pallas as pl
from jax.experimental.pallas import tpu as pltpu
from jax.experimental.pallas import tpu_sc as plsc

N = 10000
E = 320000
D = 128
ED = 16
H = 8
C = 128
HC = H * C
FF = 128

NP = 10240
R = 32
B = NP // R
CE = 128
NCHUNK = E // CE + B
EP = NCHUNK * CE
NW = 32
RPW = EP // NW
K = 16
NCK = RPW // K


def _proj_body(x_ref, wl_ref, wr_ref, xl_ref, xr_ref):
    x = x_ref[...]
    xl_ref[...] = jnp.dot(x, wl_ref[...], preferred_element_type=jnp.float32)
    xr_ref[...] = jnp.dot(x, wr_ref[...], preferred_element_type=jnp.float32)


def _project_nodes(x_p, Wl, Wr):
    return pl.pallas_call(
        _proj_body,
        grid=(NP // 256,),
        in_specs=[
            pl.BlockSpec((256, D), lambda i: (i, 0)),
            pl.BlockSpec((D, HC), lambda i: (0, 0)),
            pl.BlockSpec((D, HC), lambda i: (0, 0)),
        ],
        out_specs=[
            pl.BlockSpec((256, HC), lambda i: (i, 0)),
            pl.BlockSpec((256, HC), lambda i: (i, 0)),
        ],
        out_shape=[
            jax.ShapeDtypeStruct((NP, HC), jnp.float32),
            jax.ShapeDtypeStruct((NP, HC), jnp.float32),
        ],
    )(x_p, Wl, Wr)


def _project_edges(ea_p, We1, We2):
    return pl.pallas_call(
        _proj_body,
        grid=(EP // 1920,),
        in_specs=[
            pl.BlockSpec((1920, ED), lambda i: (i, 0)),
            pl.BlockSpec((ED, HC), lambda i: (0, 0)),
            pl.BlockSpec((ED, HC), lambda i: (0, 0)),
        ],
        out_specs=[
            pl.BlockSpec((1920, HC), lambda i: (i, 0)),
            pl.BlockSpec((1920, HC), lambda i: (i, 0)),
        ],
        out_shape=[
            jax.ShapeDtypeStruct((EP, HC), jnp.float32),
            jax.ShapeDtypeStruct((EP, HC), jnp.float32),
        ],
    )(ea_p, We1, We2)


def _edge_body(cb_ref, cf_ref, xls_ref, eaw_ref, xr_ref, dr_ref, vd_ref,
               att_ref, out_ref, den_ref):
    k = pl.program_id(0)
    xls = xls_ref[...]
    rows = lax.broadcasted_iota(jnp.int32, (CE, R), 1)
    Sf = jnp.where(dr_ref[...] == rows, 1.0, 0.0)
    xr_rows = jnp.dot(Sf, xr_ref[...], preferred_element_type=jnp.float32)
    u = xls + xr_rows + eaw_ref[...]
    u = jnp.maximum(u, 0.2 * u)
    ua = u * att_ref[...]
    ci = lax.broadcasted_iota(jnp.int32, (HC, 16), 0)
    hi = lax.broadcasted_iota(jnp.int32, (HC, 16), 1)
    sel = jnp.where(ci // C == hi, 1.0, 0.0)
    logits = jnp.dot(ua, sel, preferred_element_type=jnp.float32)
    w = jnp.exp(logits) * vd_ref[...]
    hr = lax.broadcasted_iota(jnp.int32, (16, HC), 0)
    hc = lax.broadcasted_iota(jnp.int32, (16, HC), 1)
    exp_m = jnp.where(hc // C == hr, 1.0, 0.0)
    wexp = jnp.dot(w, exp_m, preferred_element_type=jnp.float32)
    dn = ((0,), (0,)), ((), ())
    dpart = lax.dot_general(Sf, w, dn, preferred_element_type=jnp.float32)
    opart = lax.dot_general(Sf, wexp * xls, dn,
                            preferred_element_type=jnp.float32)

    @pl.when(cf_ref[k] == 1)
    def _():
        out_ref[...] = jnp.zeros_like(out_ref)
        den_ref[...] = jnp.zeros_like(den_ref)

    out_ref[...] += opart
    den_ref[...] += dpart


def _edge_pass(cb, cf, XLs, EAW, XR, dst_rel, valid, att):
    grid_spec = pltpu.PrefetchScalarGridSpec(
        num_scalar_prefetch=2,
        grid=(NCHUNK,),
        in_specs=[
            pl.BlockSpec((CE, HC), lambda k, cb, cf: (k, 0)),
            pl.BlockSpec((CE, HC), lambda k, cb, cf: (k, 0)),
            pl.BlockSpec((R, HC), lambda k, cb, cf: (cb[k], 0)),
            pl.BlockSpec((CE, 1), lambda k, cb, cf: (k, 0)),
            pl.BlockSpec((CE, 1), lambda k, cb, cf: (k, 0)),
            pl.BlockSpec((1, HC), lambda k, cb, cf: (0, 0)),
        ],
        out_specs=[
            pl.BlockSpec((R, HC), lambda k, cb, cf: (cb[k], 0)),
            pl.BlockSpec((R, 16), lambda k, cb, cf: (cb[k], 0)),
        ],
    )
    return pl.pallas_call(
        _edge_body,
        grid_spec=grid_spec,
        out_shape=[
            jax.ShapeDtypeStruct((NP, HC), jnp.float32),
            jax.ShapeDtypeStruct((NP, 16), jnp.float32),
        ],
    )(cb, cf, XLs, EAW, XR, dst_rel, valid, att)


def _finish_h(acc, den, b):
    recip = 1.0 / (den[:, :H] + 1e-16)
    rowi = lax.broadcasted_iota(jnp.int32, (H, HC), 0)
    coli = lax.broadcasted_iota(jnp.int32, (H, HC), 1)
    sel = jnp.where(coli // C == rowi, 1.0, 0.0)
    den_wide = jnp.dot(recip, sel, preferred_element_type=jnp.float32)
    return jnp.maximum(acc * den_wide + b, 0.0)


def _epi_body(acc_ref, den_ref, b_ref, wl_ref, wr_ref, xl_ref, xr_ref):
    h = _finish_h(acc_ref[...], den_ref[...], b_ref[...])
    xl_ref[...] = jnp.dot(h, wl_ref[...], preferred_element_type=jnp.float32)
    xr_ref[...] = jnp.dot(h, wr_ref[...], preferred_element_type=jnp.float32)


def _epilogue_project(acc, den, b, Wl, Wr):
    return pl.pallas_call(
        _epi_body,
        grid=(NP // 256,),
        in_specs=[
            pl.BlockSpec((256, HC), lambda i: (i, 0)),
            pl.BlockSpec((256, 16), lambda i: (i, 0)),
            pl.BlockSpec((1, HC), lambda i: (0, 0)),
            pl.BlockSpec((HC, HC), lambda i: (0, 0)),
            pl.BlockSpec((HC, HC), lambda i: (0, 0)),
        ],
        out_specs=[
            pl.BlockSpec((256, HC), lambda i: (i, 0)),
            pl.BlockSpec((256, HC), lambda i: (i, 0)),
        ],
        out_shape=[
            jax.ShapeDtypeStruct((NP, HC), jnp.float32),
            jax.ShapeDtypeStruct((NP, HC), jnp.float32),
        ],
    )(acc, den, b, Wl, Wr)


def _pool_body(acc_ref, den_ref, b_ref, out_ref):
    i = pl.program_id(0)
    h = _finish_h(acc_ref[...], den_ref[...], b_ref[...])
    rows = i * 256 + lax.broadcasted_iota(jnp.int32, (256, HC), 0)
    h = jnp.where(rows < N, h, 0.0)
    part = jnp.sum(h, axis=0, keepdims=True)

    @pl.when(i == 0)
    def _():
        out_ref[...] = jnp.zeros_like(out_ref)

    out_ref[...] += part


def _pool_partials(acc, den, b):
    return pl.pallas_call(
        _pool_body,
        grid=(NP // 256,),
        in_specs=[
            pl.BlockSpec((256, HC), lambda i: (i, 0)),
            pl.BlockSpec((256, 16), lambda i: (i, 0)),
            pl.BlockSpec((1, HC), lambda i: (0, 0)),
        ],
        out_specs=pl.BlockSpec((1, HC), lambda i: (0, 0)),
        out_shape=jax.ShapeDtypeStruct((1, HC), jnp.float32),
    )(acc, den, b)


def _head_body(p_ref, wf1_ref, bf1_ref, wf2_ref, bf2_ref, o_ref):
    pooled = p_ref[...] / float(N)
    z = jnp.dot(pooled, wf1_ref[...], preferred_element_type=jnp.float32)
    z = jnp.maximum(z + bf1_ref[...], 0.0)
    o_ref[...] = jnp.dot(z, wf2_ref[...], preferred_element_type=jnp.float32) + bf2_ref[...]


def _ffn_head(partials, Wf1, bf1, Wf2, bf2):
    return pl.pallas_call(
        _head_body,
        in_specs=[
            pl.BlockSpec(partials.shape, lambda: (0, 0)),
            pl.BlockSpec(Wf1.shape, lambda: (0, 0)),
            pl.BlockSpec((1, FF), lambda: (0, 0)),
            pl.BlockSpec(Wf2.shape, lambda: (0, 0)),
            pl.BlockSpec((1, 1), lambda: (0, 0)),
        ],
        out_specs=pl.BlockSpec((1, 1), lambda: (0, 0)),
        out_shape=jax.ShapeDtypeStruct((1, 1), jnp.float32),
    )(partials, Wf1, bf1.reshape(1, FF), Wf2, bf2.reshape(1, 1))


def _sc_gather_body(xl_hbm, src_hbm, out_hbm, idx_all, buf, sem):
    cid = lax.axis_index("c")
    sid = lax.axis_index("s")
    wid = sid * 2 + cid
    base = pl.multiple_of(wid * RPW, 16)
    pltpu.sync_copy(src_hbm.at[pl.ds(base, RPW)], idx_all)

    def _chunk(k, carry):
        sv = idx_all[pl.ds(k * K, K)]
        cp = pltpu.async_copy(xl_hbm.at[sv], buf, sem)
        cp.wait()
        e0 = pl.multiple_of(base + k * K, 16)
        pltpu.sync_copy(buf, out_hbm.at[pl.ds(e0, K)])
        return carry

    lax.fori_loop(0, NCK, _chunk, 0)


def _sc_gather(table, src_p):
    fn = pl.kernel(
        _sc_gather_body,
        out_type=jax.ShapeDtypeStruct((EP, HC), jnp.float32),
        mesh=plsc.VectorSubcoreMesh(core_axis_name="c", subcore_axis_name="s"),
        scratch_types=[
            pltpu.VMEM((RPW,), jnp.int32),
            pltpu.VMEM((K, HC), jnp.float32),
            pltpu.SemaphoreType.DMA,
        ],
    )
    return fn(table, src_p)


def _prep(src, dst, edge_attr):
    iota_e = jnp.arange(E, dtype=jnp.int32)
    dst_s, src_s, order = lax.sort((dst, src, iota_e), num_keys=1)
    ea_s = edge_attr[order]

    bnd = jnp.arange(B, dtype=jnp.int32) * R
    starts = jnp.searchsorted(dst_s, bnd).astype(jnp.int32)
    ends = jnp.searchsorted(dst_s, bnd + R).astype(jnp.int32)
    cnt = ends - starts
    chunks_b = jnp.maximum((cnt + CE - 1) // CE, 1)
    chunk_start = jnp.concatenate(
        [jnp.zeros((1,), jnp.int32), jnp.cumsum(chunks_b).astype(jnp.int32)])
    bs = dst_s // R
    slot = chunk_start[bs] * CE + (iota_e - starts[bs])
    src_p = jnp.zeros((EP,), jnp.int32).at[slot].set(src_s)
    dst_rel = jnp.zeros((EP, 1), jnp.int32).at[slot, 0].set(dst_s - bs * R)
    valid = jnp.zeros((EP, 1), jnp.float32).at[slot, 0].set(1.0)
    ea_p = jnp.zeros((EP, ED), jnp.float32).at[slot].set(ea_s)
    ck = jnp.arange(NCHUNK, dtype=jnp.int32)
    cb = jnp.clip(
        jnp.searchsorted(chunk_start, ck, side="right").astype(jnp.int32) - 1,
        0, B - 1)
    cf = (ck == chunk_start[cb]).astype(jnp.int32)
    return src_p, dst_rel, valid, ea_p, cb, cf


def kernel(x, edge_index, edge_attr, batch, Wl1, Wr1, We1, att1, b1,
           Wl2, Wr2, We2, att2, b2, Wf1, bf1, Wf2, bf2):
    src_p, dst_rel, valid, ea_p, cb, cf = _prep(
        edge_index[0], edge_index[1], edge_attr)

    x_p = jnp.pad(x, ((0, NP - N), (0, 0)))

    XL1, XR1 = _project_nodes(x_p, Wl1, Wr1)
    EAW1, EAW2 = _project_edges(ea_p, We1, We2)
    att1f = att1.reshape(1, HC)
    att2f = att2.reshape(1, HC)

    XLs1 = _sc_gather(XL1, src_p)
    out1, den1 = _edge_pass(cb, cf, XLs1, EAW1, XR1, dst_rel, valid, att1f)
    XL2, XR2 = _epilogue_project(out1, den1, b1.reshape(1, HC), Wl2, Wr2)
    XLs2 = _sc_gather(XL2, src_p)
    out2, den2 = _edge_pass(cb, cf, XLs2, EAW2, XR2, dst_rel, valid, att2f)
    partials = _pool_partials(out2, den2, b2.reshape(1, HC))
    return _ffn_head(partials, Wf1, bf1, Wf2, bf2)


# speedup vs baseline: 4.5792x; 1.0541x over previous
import functools

import jax
import jax.numpy as jnp
from jax import lax
from jax.experimental import pallas as pl
from jax.experimental.pallas import tpu as pltpu
from jax.experimental.pallas import tpu_sc as plsc

N = 10000
E = 320000
D = 128
ED = 16
H = 8
C = 128
HC = H * C
FF = 128

NP = 10240
R = 32
B = NP // R
CE = 128
NCHUNK = E // CE + B
EP = NCHUNK * CE
NW = 32
RPW = EP // NW
K = 16
NCK = RPW // K


def _proj_body(x_ref, wl_ref, wr_ref, xl_ref, xr_ref):
    x = x_ref[...]
    xl_ref[...] = jnp.dot(x, wl_ref[...], preferred_element_type=jnp.float32)
    xr_ref[...] = jnp.dot(x, wr_ref[...], preferred_element_type=jnp.float32)


def _project_nodes(x_p, Wl, Wr):
    return pl.pallas_call(
        _proj_body,
        grid=(NP // 256,),
        in_specs=[
            pl.BlockSpec((256, D), lambda i: (i, 0)),
            pl.BlockSpec((D, HC), lambda i: (0, 0)),
            pl.BlockSpec((D, HC), lambda i: (0, 0)),
        ],
        out_specs=[
            pl.BlockSpec((256, HC), lambda i: (i, 0)),
            pl.BlockSpec((256, HC), lambda i: (i, 0)),
        ],
        out_shape=[
            jax.ShapeDtypeStruct((NP, HC), jnp.float32),
            jax.ShapeDtypeStruct((NP, HC), jnp.float32),
        ],
    )(x_p, Wl, Wr)


def _edge_body(cb_ref, cf_ref, xls_ref, ea_ref, we_ref, xr_ref, dr_ref,
               vd_ref, att_ref, out_ref, den_ref):
    k = pl.program_id(0)
    xls = xls_ref[...]
    rows = lax.broadcasted_iota(jnp.int32, (CE, R), 1)
    Sf = jnp.where(dr_ref[...] == rows, 1.0, 0.0)
    xr_rows = jnp.dot(Sf, xr_ref[...], preferred_element_type=jnp.float32)
    eaw = jnp.dot(ea_ref[...], we_ref[...], preferred_element_type=jnp.float32)
    u = xls + xr_rows + eaw
    u = jnp.maximum(u, 0.2 * u)
    ua = u * att_ref[...]
    ci = lax.broadcasted_iota(jnp.int32, (HC, 16), 0)
    hi = lax.broadcasted_iota(jnp.int32, (HC, 16), 1)
    sel = jnp.where(ci // C == hi, 1.0, 0.0)
    logits = jnp.dot(ua, sel, preferred_element_type=jnp.float32)
    w = jnp.exp(logits) * vd_ref[...]
    hr = lax.broadcasted_iota(jnp.int32, (16, HC), 0)
    hc = lax.broadcasted_iota(jnp.int32, (16, HC), 1)
    exp_m = jnp.where(hc // C == hr, 1.0, 0.0)
    wexp = jnp.dot(w, exp_m, preferred_element_type=jnp.float32)
    dn = ((0,), (0,)), ((), ())
    dpart = lax.dot_general(Sf, w, dn, preferred_element_type=jnp.float32)
    opart = lax.dot_general(Sf, wexp * xls, dn,
                            preferred_element_type=jnp.float32)

    @pl.when(cf_ref[k] == 1)
    def _():
        out_ref[...] = jnp.zeros_like(out_ref)
        den_ref[...] = jnp.zeros_like(den_ref)

    out_ref[...] += opart
    den_ref[...] += dpart


def _edge_pass(cb, cf, XLs, ea_p, We, XR, dst_rel, valid, att):
    grid_spec = pltpu.PrefetchScalarGridSpec(
        num_scalar_prefetch=2,
        grid=(NCHUNK,),
        in_specs=[
            pl.BlockSpec((CE, HC), lambda k, cb, cf: (k, 0)),
            pl.BlockSpec((CE, ED), lambda k, cb, cf: (k, 0)),
            pl.BlockSpec((ED, HC), lambda k, cb, cf: (0, 0)),
            pl.BlockSpec((R, HC), lambda k, cb, cf: (cb[k], 0)),
            pl.BlockSpec((CE, 1), lambda k, cb, cf: (k, 0)),
            pl.BlockSpec((CE, 1), lambda k, cb, cf: (k, 0)),
            pl.BlockSpec((1, HC), lambda k, cb, cf: (0, 0)),
        ],
        out_specs=[
            pl.BlockSpec((R, HC), lambda k, cb, cf: (cb[k], 0)),
            pl.BlockSpec((R, 16), lambda k, cb, cf: (cb[k], 0)),
        ],
    )
    return pl.pallas_call(
        _edge_body,
        grid_spec=grid_spec,
        out_shape=[
            jax.ShapeDtypeStruct((NP, HC), jnp.float32),
            jax.ShapeDtypeStruct((NP, 16), jnp.float32),
        ],
    )(cb, cf, XLs, ea_p, We, XR, dst_rel, valid, att)


def _finish_h(acc, den, b):
    recip = 1.0 / (den[:, :H] + 1e-16)
    rowi = lax.broadcasted_iota(jnp.int32, (H, HC), 0)
    coli = lax.broadcasted_iota(jnp.int32, (H, HC), 1)
    sel = jnp.where(coli // C == rowi, 1.0, 0.0)
    den_wide = jnp.dot(recip, sel, preferred_element_type=jnp.float32)
    return jnp.maximum(acc * den_wide + b, 0.0)


def _epi_body(acc_ref, den_ref, b_ref, wl_ref, wr_ref, xl_ref, xr_ref):
    h = _finish_h(acc_ref[...], den_ref[...], b_ref[...])
    xl_ref[...] = jnp.dot(h, wl_ref[...], preferred_element_type=jnp.float32)
    xr_ref[...] = jnp.dot(h, wr_ref[...], preferred_element_type=jnp.float32)


def _epilogue_project(acc, den, b, Wl, Wr):
    return pl.pallas_call(
        _epi_body,
        grid=(NP // 256,),
        in_specs=[
            pl.BlockSpec((256, HC), lambda i: (i, 0)),
            pl.BlockSpec((256, 16), lambda i: (i, 0)),
            pl.BlockSpec((1, HC), lambda i: (0, 0)),
            pl.BlockSpec((HC, HC), lambda i: (0, 0)),
            pl.BlockSpec((HC, HC), lambda i: (0, 0)),
        ],
        out_specs=[
            pl.BlockSpec((256, HC), lambda i: (i, 0)),
            pl.BlockSpec((256, HC), lambda i: (i, 0)),
        ],
        out_shape=[
            jax.ShapeDtypeStruct((NP, HC), jnp.float32),
            jax.ShapeDtypeStruct((NP, HC), jnp.float32),
        ],
    )(acc, den, b, Wl, Wr)


def _pool_body(acc_ref, den_ref, b_ref, out_ref):
    i = pl.program_id(0)
    h = _finish_h(acc_ref[...], den_ref[...], b_ref[...])
    rows = i * 256 + lax.broadcasted_iota(jnp.int32, (256, HC), 0)
    h = jnp.where(rows < N, h, 0.0)
    part = jnp.sum(h, axis=0, keepdims=True)

    @pl.when(i == 0)
    def _():
        out_ref[...] = jnp.zeros_like(out_ref)

    out_ref[...] += part


def _pool_partials(acc, den, b):
    return pl.pallas_call(
        _pool_body,
        grid=(NP // 256,),
        in_specs=[
            pl.BlockSpec((256, HC), lambda i: (i, 0)),
            pl.BlockSpec((256, 16), lambda i: (i, 0)),
            pl.BlockSpec((1, HC), lambda i: (0, 0)),
        ],
        out_specs=pl.BlockSpec((1, HC), lambda i: (0, 0)),
        out_shape=jax.ShapeDtypeStruct((1, HC), jnp.float32),
    )(acc, den, b)


def _head_body(p_ref, wf1_ref, bf1_ref, wf2_ref, bf2_ref, o_ref):
    pooled = p_ref[...] / float(N)
    z = jnp.dot(pooled, wf1_ref[...], preferred_element_type=jnp.float32)
    z = jnp.maximum(z + bf1_ref[...], 0.0)
    o_ref[...] = jnp.dot(z, wf2_ref[...], preferred_element_type=jnp.float32) + bf2_ref[...]


def _ffn_head(partials, Wf1, bf1, Wf2, bf2):
    return pl.pallas_call(
        _head_body,
        in_specs=[
            pl.BlockSpec(partials.shape, lambda: (0, 0)),
            pl.BlockSpec(Wf1.shape, lambda: (0, 0)),
            pl.BlockSpec((1, FF), lambda: (0, 0)),
            pl.BlockSpec(Wf2.shape, lambda: (0, 0)),
            pl.BlockSpec((1, 1), lambda: (0, 0)),
        ],
        out_specs=pl.BlockSpec((1, 1), lambda: (0, 0)),
        out_shape=jax.ShapeDtypeStruct((1, 1), jnp.float32),
    )(partials, Wf1, bf1.reshape(1, FF), Wf2, bf2.reshape(1, 1))


def _sc_gather_body(xl_hbm, src_hbm, out_hbm, idx_all, buf, sem):
    cid = lax.axis_index("c")
    sid = lax.axis_index("s")
    wid = sid * 2 + cid
    base = pl.multiple_of(wid * RPW, 16)
    pltpu.sync_copy(src_hbm.at[pl.ds(base, RPW)], idx_all)

    def _chunk(k, carry):
        sv = idx_all[pl.ds(k * K, K)]
        cp = pltpu.async_copy(xl_hbm.at[sv], buf, sem)
        cp.wait()
        e0 = pl.multiple_of(base + k * K, 16)
        pltpu.sync_copy(buf, out_hbm.at[pl.ds(e0, K)])
        return carry

    lax.fori_loop(0, NCK, _chunk, 0)


def _sc_gather(table, src_p):
    fn = pl.kernel(
        _sc_gather_body,
        out_type=jax.ShapeDtypeStruct((EP, HC), jnp.float32),
        mesh=plsc.VectorSubcoreMesh(core_axis_name="c", subcore_axis_name="s"),
        scratch_types=[
            pltpu.VMEM((RPW,), jnp.int32),
            pltpu.VMEM((K, HC), jnp.float32),
            pltpu.SemaphoreType.DMA,
        ],
    )
    return fn(table, src_p)


def _prep(src, dst, edge_attr):
    iota_e = jnp.arange(E, dtype=jnp.int32)
    dst_s, src_s, order = lax.sort((dst, src, iota_e), num_keys=1)
    ea_s = edge_attr[order]

    bnd = jnp.arange(B, dtype=jnp.int32) * R
    starts = jnp.searchsorted(dst_s, bnd).astype(jnp.int32)
    ends = jnp.searchsorted(dst_s, bnd + R).astype(jnp.int32)
    cnt = ends - starts
    chunks_b = jnp.maximum((cnt + CE - 1) // CE, 1)
    chunk_start = jnp.concatenate(
        [jnp.zeros((1,), jnp.int32), jnp.cumsum(chunks_b).astype(jnp.int32)])
    bs = dst_s // R
    slot = chunk_start[bs] * CE + (iota_e - starts[bs])
    src_p = jnp.zeros((EP,), jnp.int32).at[slot].set(src_s)
    dst_rel = jnp.zeros((EP, 1), jnp.int32).at[slot, 0].set(dst_s - bs * R)
    valid = jnp.zeros((EP, 1), jnp.float32).at[slot, 0].set(1.0)
    ea_p = jnp.zeros((EP, ED), jnp.float32).at[slot].set(ea_s)
    ck = jnp.arange(NCHUNK, dtype=jnp.int32)
    cb = jnp.clip(
        jnp.searchsorted(chunk_start, ck, side="right").astype(jnp.int32) - 1,
        0, B - 1)
    cf = (ck == chunk_start[cb]).astype(jnp.int32)
    return src_p, dst_rel, valid, ea_p, cb, cf


def kernel(x, edge_index, edge_attr, batch, Wl1, Wr1, We1, att1, b1,
           Wl2, Wr2, We2, att2, b2, Wf1, bf1, Wf2, bf2):
    src_p, dst_rel, valid, ea_p, cb, cf = _prep(
        edge_index[0], edge_index[1], edge_attr)

    x_p = jnp.pad(x, ((0, NP - N), (0, 0)))

    XL1, XR1 = _project_nodes(x_p, Wl1, Wr1)
    att1f = att1.reshape(1, HC)
    att2f = att2.reshape(1, HC)

    XLs1 = _sc_gather(XL1, src_p)
    out1, den1 = _edge_pass(cb, cf, XLs1, ea_p, We1, XR1, dst_rel, valid,
                            att1f)
    XL2, XR2 = _epilogue_project(out1, den1, b1.reshape(1, HC), Wl2, Wr2)
    XLs2 = _sc_gather(XL2, src_p)
    out2, den2 = _edge_pass(cb, cf, XLs2, ea_p, We2, XR2, dst_rel, valid,
                            att2f)
    partials = _pool_partials(out2, den2, b2.reshape(1, HC))
    return _ffn_head(partials, Wf1, bf1, Wf2, bf2)
